# Initial kernel scaffold; baseline (speedup 1.0000x reference)
import jax
import jax.numpy as jnp
from jax.experimental import pallas as pl


def kernel(x, edge_index, W1, b1, W2, b2, W3, b3):
    raise NotImplementedError("write your pallas kernel here")


# speedup vs baseline: 10.4504x; 10.4504x over previous
import functools

import jax
import jax.numpy as jnp
from jax import lax
from jax.experimental import pallas as pl
from jax.experimental.pallas import tpu as pltpu
from jax.experimental.pallas import tpu_sc as plsc

N_NODES = 10000
D = 128
N_EDGES = 320000

NC = 2
NS = 16
NW = NC * NS
K = 128
CHUNKS = 79
E_PAD = NW * CHUNKS * K
ACC_ROWS = 10240
TRASH_ROW = N_NODES
ZROWS = ACC_ROWS // NS
OROWS = 624
OTAIL = N_NODES - NS * OROWS

_mesh = plsc.VectorSubcoreMesh(core_axis_name="c", subcore_axis_name="s")


@functools.partial(
    pl.kernel,
    mesh=_mesh,
    out_type=jax.ShapeDtypeStruct((NC, N_NODES, 8), jnp.float32),
    scratch_types=[
        pltpu.VMEM((CHUNKS, K), jnp.int32),
        pltpu.VMEM((K, 8), jnp.float32),
        pltpu.VMEM_SHARED((ACC_ROWS, 8), jnp.float32),
        pltpu.SemaphoreType.DMA,
    ],
)
def _deg_kernel(dst_hbm, zeros_hbm, ones_hbm, out_hbm, dst_v, ones_v, acc, sem):
    c = lax.axis_index("c")
    s = lax.axis_index("s")
    wid = s * NC + c
    pltpu.sync_copy(zeros_hbm, acc.at[pl.ds(s * ZROWS, ZROWS)])
    pltpu.sync_copy(ones_hbm, ones_v)
    pltpu.sync_copy(dst_hbm.at[wid], dst_v)
    plsc.subcore_barrier()

    def body(j, carry):
        pltpu.sync_copy(ones_v, acc.at[dst_v.at[j]], add=True)
        return carry

    lax.fori_loop(0, CHUNKS, body, 0)
    plsc.subcore_barrier()
    pltpu.sync_copy(acc.at[pl.ds(s * OROWS, OROWS)],
                    out_hbm.at[c, pl.ds(s * OROWS, OROWS)])

    @pl.when(s == NS - 1)
    def _():
        pltpu.sync_copy(acc.at[pl.ds(NS * OROWS, OTAIL)],
                        out_hbm.at[c, pl.ds(NS * OROWS, OTAIL)])


@functools.partial(
    pl.kernel,
    mesh=_mesh,
    out_type=jax.ShapeDtypeStruct((NC, N_NODES, D), jnp.float32),
    scratch_types=[
        pltpu.VMEM((CHUNKS, K), jnp.int32),
        pltpu.VMEM((CHUNKS, K), jnp.int32),
        pltpu.VMEM((K, D), jnp.float32),
        pltpu.VMEM_SHARED((ACC_ROWS, D), jnp.float32),
        pltpu.SemaphoreType.DMA,
    ],
)
def _agg_kernel(hs_hbm, src_hbm, dst_hbm, zeros_hbm, out_hbm,
                src_v, dst_v, rows_v, acc, sem):
    c = lax.axis_index("c")
    s = lax.axis_index("s")
    wid = s * NC + c
    pltpu.sync_copy(zeros_hbm, acc.at[pl.ds(s * ZROWS, ZROWS)])
    pltpu.sync_copy(src_hbm.at[wid], src_v)
    pltpu.sync_copy(dst_hbm.at[wid], dst_v)
    plsc.subcore_barrier()

    def body(j, carry):
        pltpu.async_copy(hs_hbm.at[src_v.at[j]], rows_v, sem).wait()
        pltpu.sync_copy(rows_v, acc.at[dst_v.at[j]], add=True)
        return carry

    lax.fori_loop(0, CHUNKS, body, 0)
    plsc.subcore_barrier()
    pltpu.sync_copy(acc.at[pl.ds(s * OROWS, OROWS)],
                    out_hbm.at[c, pl.ds(s * OROWS, OROWS)])

    @pl.when(s == NS - 1)
    def _():
        pltpu.sync_copy(acc.at[pl.ds(NS * OROWS, OTAIL)],
                        out_hbm.at[c, pl.ds(NS * OROWS, OTAIL)])


_R = 1000


def _dinv_of(deg8a, deg8b):
    deg = jnp.sum(deg8a + deg8b, axis=1, keepdims=True) + 1.0
    return lax.rsqrt(deg)


def _mm_first_body(x_ref, w_ref, da_ref, db_ref, o_ref):
    dinv = _dinv_of(da_ref[...], db_ref[...])
    o_ref[...] = jnp.dot(x_ref[...], w_ref[...],
                         preferred_element_type=jnp.float32) * dinv


def _mm_mid_body(p0_ref, p1_ref, hs_ref, da_ref, db_ref, b_ref, w_ref, o_ref):
    dinv = _dinv_of(da_ref[...], db_ref[...])
    t = dinv * (p0_ref[...] + p1_ref[...] + hs_ref[...]) + b_ref[...]
    t = jnp.maximum(t, 0.0)
    o_ref[...] = jnp.dot(t, w_ref[...],
                         preferred_element_type=jnp.float32) * dinv


def _final_body(p0_ref, p1_ref, hs_ref, da_ref, db_ref, b_ref, o_ref):
    dinv = _dinv_of(da_ref[...], db_ref[...])
    o_ref[...] = dinv * (p0_ref[...] + p1_ref[...] + hs_ref[...]) + b_ref[...]


_row_spec = pl.BlockSpec((_R, D), lambda i: (i, 0))
_d8_spec = pl.BlockSpec((_R, 8), lambda i: (i, 0))
_w_spec = pl.BlockSpec((D, D), lambda i: (0, 0))
_b_spec = pl.BlockSpec((1, D), lambda i: (0, 0))
_out_sds = jax.ShapeDtypeStruct((N_NODES, D), jnp.float32)
_grid = (N_NODES // _R,)

_mm_first = pl.pallas_call(
    _mm_first_body, grid=_grid,
    in_specs=[_row_spec, _w_spec, _d8_spec, _d8_spec],
    out_specs=_row_spec, out_shape=_out_sds)

_mm_mid = pl.pallas_call(
    _mm_mid_body, grid=_grid,
    in_specs=[_row_spec, _row_spec, _row_spec, _d8_spec, _d8_spec,
              _b_spec, _w_spec],
    out_specs=_row_spec, out_shape=_out_sds)

_final = pl.pallas_call(
    _final_body, grid=_grid,
    in_specs=[_row_spec, _row_spec, _row_spec, _d8_spec, _d8_spec, _b_spec],
    out_specs=_row_spec, out_shape=_out_sds)


def kernel(x, edge_index, W1, b1, W2, b2, W3, b3):
    src = edge_index[0].astype(jnp.int32)
    dst = edge_index[1].astype(jnp.int32)
    pad = E_PAD - N_EDGES
    srcp = jnp.concatenate([src, jnp.zeros((pad,), jnp.int32)])
    dstp = jnp.concatenate([dst, jnp.full((pad,), TRASH_ROW, jnp.int32)])
    srcp = srcp.reshape(NW, CHUNKS, K)
    dstp = dstp.reshape(NW, CHUNKS, K)

    zeros8 = jnp.zeros((ZROWS, 8), jnp.float32)
    ones8 = jnp.ones((K, 8), jnp.float32)
    zerosD = jnp.zeros((ZROWS, D), jnp.float32)

    degp = _deg_kernel(dstp, zeros8, ones8)
    da, db = degp[0], degp[1]
    b1r = b1.reshape(1, D)
    b2r = b2.reshape(1, D)
    b3r = b3.reshape(1, D)

    hs1 = _mm_first(x, W1, da, db)
    a1 = _agg_kernel(hs1, srcp, dstp, zerosD)
    hs2 = _mm_mid(a1[0], a1[1], hs1, da, db, b1r, W2)
    a2 = _agg_kernel(hs2, srcp, dstp, zerosD)
    hs3 = _mm_mid(a2[0], a2[1], hs2, da, db, b2r, W3)
    a3 = _agg_kernel(hs3, srcp, dstp, zerosD)
    return _final(a3[0], a3[1], hs3, da, db, b3r)
